# baseline (speedup 1.0000x reference)
import functools

import jax
import jax.numpy as jnp
from jax import lax
from jax.experimental import pallas as pl
from jax.experimental.pallas import tpu as pltpu
from jax.experimental.pallas import tpu_sc as plsc

N_NODES = 10000
N_IN = 128
HEADS = 8
CH = 16
HC = HEADS * CH
E_EDGES = 320000
E_TOT = E_EDGES + N_NODES

LANES = 16
NC = 2
NS = 16
NW = NC * NS

K_CHUNK = 64
EW = 10368
EP = EW * NW
N_CHUNKS = EW // K_CHUNK
NP = 10016
ROWS_PER_SUB = NP // NS

_SPLAT_DNUMS = lax.GatherDimensionNumbers(
    offset_dims=(), collapsed_slice_dims=(0,), start_index_map=(0,))


def _splat(vec, lane):
  idx = jnp.full((LANES, 1), lane, jnp.int32)
  return lax.gather(vec, idx, _SPLAT_DNUMS, (1,),
                    mode=lax.GatherScatterMode.PROMISE_IN_BOUNDS)


def _make_sc_edge_kernel(d_model, hs_w, hoff, heads):
  mesh = plsc.VectorSubcoreMesh(core_axis_name="c", subcore_axis_name="s")
  nch2 = N_CHUNKS // 2

  @functools.partial(
      pl.kernel,
      out_type=jax.ShapeDtypeStruct((NC, NP, hs_w), jnp.float32),
      mesh=mesh,
      compiler_params=pltpu.CompilerParams(use_tc_tiling_on_sc=False),
      scratch_types=[
          pltpu.VMEM((2, K_CHUNK), jnp.int32),
          pltpu.VMEM((2, K_CHUNK), jnp.int32),
          pltpu.VMEM((2, K_CHUNK, hs_w), jnp.float32),
          pltpu.VMEM((2, K_CHUNK, LANES), jnp.float32),
          pltpu.VMEM((2, K_CHUNK, hs_w), jnp.float32),
          pltpu.VMEM_SHARED((NP, hs_w), jnp.float32),
          pltpu.SemaphoreType.DMA,
          pltpu.SemaphoreType.DMA,
          pltpu.SemaphoreType.DMA,
          pltpu.SemaphoreType.DMA,
      ],
  )
  def sc_kernel(hs_hbm, adst_hbm, src_hbm, dst_hbm, acc_out,
                sidx, didx, hsb, adb, mbw, acc_sh,
                gsem0, gsem1, ssem0, ssem1):
    c = lax.axis_index("c")
    s = lax.axis_index("s")
    wid = s * NC + c
    wgroups = hs_w // LANES

    zero16 = jnp.zeros((LANES,), jnp.float32)

    def zero_body(i, _):
      for g in range(wgroups):
        mbw[0, i, g * LANES:(g + 1) * LANES] = zero16
      return 0

    lax.fori_loop(0, K_CHUNK, zero_body, 0)
    zbase = s * ROWS_PER_SUB
    for t in range(ROWS_PER_SUB // K_CHUNK):
      pltpu.sync_copy(mbw.at[0],
                      acc_sh.at[pl.ds(zbase + t * K_CHUNK, K_CHUNK)])
    rem = ROWS_PER_SUB % K_CHUNK
    if rem:
      rb = zbase + (ROWS_PER_SUB // K_CHUNK) * K_CHUNK
      pltpu.sync_copy(mbw.at[0, pl.ds(0, rem)], acc_sh.at[pl.ds(rb, rem)])
    plsc.subcore_barrier()

    def start_gather(b):
      gs = gsem0 if b == 0 else gsem1
      g1 = pltpu.async_copy(hs_hbm.at[sidx.at[b]], hsb.at[b], gs)
      g2 = pltpu.async_copy(adst_hbm.at[didx.at[b]], adb.at[b], gs)
      return g1, g2

    def start_scatter(b):
      ss = ssem0 if b == 0 else ssem1
      return pltpu.async_copy(mbw.at[b], acc_sh.at[didx.at[b]], ss,
                              add=True)

    def compute(b):
      def edge_body(k4, _):
        for u in range(4):
          k = 4 * k4 + u
          a = hsb[b, k, hoff:hoff + LANES]
          bb = adb[b, k, :]
          e = a + bb
          e = jnp.maximum(e, 0.2 * e)
          w = jnp.exp(e)
          mbw[b, k, d_model:d_model + LANES] = w
          for h in range(heads):
            ws = _splat(w, h)
            mbw[b, k, h * LANES:(h + 1) * LANES] = (
                hsb[b, k, h * LANES:(h + 1) * LANES] * ws)
        return 0

      lax.fori_loop(0, K_CHUNK // 4, edge_body, 0)

    def chunk_body(j, _):
      row = wid * N_CHUNKS + 2 * j
      pltpu.sync_copy(src_hbm.at[pl.ds(row, 2)], sidx)
      pltpu.sync_copy(dst_hbm.at[pl.ds(row, 2)], didx)
      g0 = start_gather(0)
      g1 = start_gather(1)
      for g in g0:
        g.wait()
      compute(0)
      s0 = start_scatter(0)
      for g in g1:
        g.wait()
      compute(1)
      s1 = start_scatter(1)
      s0.wait()
      s1.wait()
      return 0

    lax.fori_loop(0, nch2, chunk_body, 0)
    plsc.subcore_barrier()
    pltpu.sync_copy(acc_sh.at[pl.ds(zbase, ROWS_PER_SUB)],
                    acc_out.at[c, pl.ds(zbase, ROWS_PER_SUB)])

  return sc_kernel


_sc_edge_big = _make_sc_edge_kernel(HC, HC + LANES, HC, HEADS)
_sc_edge_small = _make_sc_edge_kernel(CH, 2 * CH, CH, 1)


def _group_mat(rows, cols):
  r = lax.broadcasted_iota(jnp.int32, (rows, cols), 0)
  c = lax.broadcasted_iota(jnp.int32, (rows, cols), 1)
  return (r // LANES == c).astype(jnp.float32)


def _group_mat_t(rows, cols):
  r = lax.broadcasted_iota(jnp.int32, (rows, cols), 0)
  c = lax.broadcasted_iota(jnp.int32, (rows, cols), 1)
  return (c // LANES == r).astype(jnp.float32)


def _dot(a, b):
  return jnp.dot(a, b, preferred_element_type=jnp.float32,
                 precision=lax.Precision.HIGHEST)


def _head_coeffs(h, af_src, af_dst):
  g = _group_mat(h.shape[1], LANES)
  return _dot(h * af_src, g), _dot(h * af_dst, g)


RB = 2000
N_RB = N_NODES // RB


def _tc1_body(x_ref, w1_ref, afs_ref, afd_ref, ws1_ref, bs1_ref,
              h1_ref, as16_ref, ad16_ref, xin_ref):
  x = x_ref[...]
  h1 = _dot(x, w1_ref[...])
  h1_ref[...] = h1
  as16, ad16 = _head_coeffs(h1, afs_ref[...], afd_ref[...])
  as16_ref[...] = as16
  ad16_ref[...] = ad16
  xin_ref[...] = _dot(x, ws1_ref[...]) + bs1_ref[...]


def _finalize_gat(acc_ref, b_ref):
  a = acc_ref[0] + acc_ref[1]
  d = a.shape[1] - LANES
  msg = a[:, :d]
  den16 = a[:, d:]
  dbn = _dot(den16, _group_mat_t(LANES, d))
  return msg / (dbn + 1e-16) + b_ref[...]


def _gat_stats_body(acc_ref, b_ref, gat_ref, sum_ref, sq_ref):
  gat = _finalize_gat(acc_ref, b_ref)
  gat_ref[...] = gat

  @pl.when(pl.program_id(0) == 0)
  def _init():
    sum_ref[...] = jnp.zeros_like(sum_ref)
    sq_ref[...] = jnp.zeros_like(sq_ref)

  sum_ref[0:1, :] += jnp.sum(gat, axis=0, keepdims=True)
  sq_ref[0:1, :] += jnp.sum(gat * gat, axis=0, keepdims=True)


def _bn_elu(gat_ref, sum_ref, sq_ref, g_ref, be_ref):
  mean = sum_ref[0:1, :] * (1.0 / N_NODES)
  var = sq_ref[0:1, :] * (1.0 / N_NODES) - mean * mean
  y = (gat_ref[...] - mean) * lax.rsqrt(var + 1e-5) * g_ref[...] + be_ref[...]
  return jnp.where(y > 0, y, jnp.exp(y) - 1.0)


def _tc2_body(gat_ref, sum_ref, sq_ref, g1_ref, be1_ref, xin_ref, w2_ref,
              afs_ref, afd_ref, ws2_ref, bs2_ref,
              h2_ref, as16_ref, ad16_ref, xskip_ref):
  t2 = _bn_elu(gat_ref, sum_ref, sq_ref, g1_ref, be1_ref) + xin_ref[...]
  h2 = _dot(t2, w2_ref[...])
  h2_ref[...] = h2
  as16, ad16 = _head_coeffs(h2, afs_ref[...], afd_ref[...])
  as16_ref[...] = as16
  ad16_ref[...] = ad16
  xskip_ref[...] = _dot(t2, ws2_ref[...]) + bs2_ref[...]


def _tc3_body(gat_ref, sum_ref, sq_ref, g2_ref, be2_ref, w3_ref,
              as3_ref, ad3_ref, h3_ref, as16_ref, ad16_ref):
  t3 = _bn_elu(gat_ref, sum_ref, sq_ref, g2_ref, be2_ref)
  h3 = _dot(t3, w3_ref[...])
  h3_ref[...] = h3
  lane0 = (lax.broadcasted_iota(jnp.int32, (CH, CH), 1) == 0
           ).astype(jnp.float32)
  as16_ref[...] = _dot(h3, jnp.reshape(as3_ref[...], (CH, 1)) * lane0)
  ad16_ref[...] = _dot(h3, jnp.reshape(ad3_ref[...], (CH, 1)) * lane0)


def _tc4_body(acc_ref, b3_ref, xskip_ref, out_ref):
  a = acc_ref[0] + acc_ref[1]
  den = a[:, CH:CH + 1]
  out_ref[...] = a[:, :CH] / (den + 1e-16) + b3_ref[...] + xskip_ref[...]


def _sds(shape):
  return jax.ShapeDtypeStruct(shape, jnp.float32)


def _pad_nodes(a):
  return jnp.pad(a, ((0, NP - N_NODES), (0, 0)))


def kernel(x, edge_index, W1, a_src1, a_dst1, b1, g1, be1,
           W2, a_src2, a_dst2, b2, g2, be2, W3, a_src3, a_dst3, b3,
           Ws1, bs1, Ws2, bs2):
  loop = jnp.arange(N_NODES, dtype=jnp.int32)
  pad = EP - E_TOT
  src = jnp.concatenate(
      [edge_index[0], loop, jnp.zeros((pad,), jnp.int32)]
  ).reshape(EP // K_CHUNK, K_CHUNK)
  dst = jnp.concatenate(
      [edge_index[1], loop, jnp.full((pad,), N_NODES, jnp.int32)]
  ).reshape(EP // K_CHUNK, K_CHUNK)

  r1 = lambda a: jnp.reshape(a, (1, -1))

  def rows(w):
    return pl.BlockSpec((RB, w), lambda i: (i, 0))

  def full(shape):
    nd = len(shape)
    return pl.BlockSpec(shape, lambda i: (0,) * nd)

  def parts(w):
    return pl.BlockSpec((NC, RB, w), lambda i: (0, i, 0))

  h1, as16, ad16, xin = pl.pallas_call(
      _tc1_body,
      grid=(N_RB,),
      in_specs=[rows(N_IN), full((N_IN, HC)), full((1, HC)), full((1, HC)),
                full((N_IN, HC)), full((1, HC))],
      out_specs=[rows(HC), rows(LANES), rows(LANES), rows(HC)],
      out_shape=[_sds((N_NODES, HC)), _sds((N_NODES, LANES)),
                 _sds((N_NODES, LANES)), _sds((N_NODES, HC))],
  )(x, W1, r1(a_src1), r1(a_dst1), Ws1, r1(bs1))

  hs1 = _pad_nodes(jnp.concatenate([h1, as16], axis=1))
  acc1 = _sc_edge_big(hs1, _pad_nodes(ad16), src, dst)

  def gat_stats(acc, b, w):
    return pl.pallas_call(
        _gat_stats_body,
        grid=(N_RB,),
        in_specs=[parts(w + LANES), full((1, w))],
        out_specs=[rows(w), full((8, w)), full((8, w))],
        out_shape=[_sds((N_NODES, w)), _sds((8, w)), _sds((8, w))],
    )(acc, b)

  gat1, sum1, sq1 = gat_stats(acc1, r1(b1), HC)

  h2, as16b, ad16b, xskip = pl.pallas_call(
      _tc2_body,
      grid=(N_RB,),
      in_specs=[rows(HC), full((8, HC)), full((8, HC)), full((1, HC)),
                full((1, HC)), rows(HC), full((HC, HC)), full((1, HC)),
                full((1, HC)), full((HC, CH)), full((1, CH))],
      out_specs=[rows(HC), rows(LANES), rows(LANES), rows(CH)],
      out_shape=[_sds((N_NODES, HC)), _sds((N_NODES, LANES)),
                 _sds((N_NODES, LANES)), _sds((N_NODES, CH))],
  )(gat1, sum1, sq1, r1(g1), r1(be1), xin, W2, r1(a_src2), r1(a_dst2),
    Ws2, r1(bs2))

  hs2 = _pad_nodes(jnp.concatenate([h2, as16b], axis=1))
  acc2 = _sc_edge_big(hs2, _pad_nodes(ad16b), src, dst)

  gat2, sum2, sq2 = gat_stats(acc2, r1(b2), HC)

  h3, as16c, ad16c = pl.pallas_call(
      _tc3_body,
      grid=(N_RB,),
      in_specs=[rows(HC), full((8, HC)), full((8, HC)), full((1, HC)),
                full((1, HC)), full((HC, CH)), full((1, CH)), full((1, CH))],
      out_specs=[rows(CH), rows(LANES), rows(LANES)],
      out_shape=[_sds((N_NODES, CH)), _sds((N_NODES, LANES)),
                 _sds((N_NODES, LANES))],
  )(gat2, sum2, sq2, r1(g2), r1(be2), W3, r1(a_src3), r1(a_dst3))

  hs3 = _pad_nodes(jnp.concatenate([h3, as16c], axis=1))
  acc3 = _sc_edge_small(hs3, _pad_nodes(ad16c), src, dst)

  out = pl.pallas_call(
      _tc4_body,
      grid=(N_RB,),
      in_specs=[parts(CH + LANES), full((1, CH)), rows(CH)],
      out_specs=rows(CH),
      out_shape=_sds((N_NODES, CH)),
  )(acc3, r1(b3), xskip)
  return out


# speedup vs baseline: 1.5643x; 1.5643x over previous
import functools

import jax
import jax.numpy as jnp
from jax import lax
from jax.experimental import pallas as pl
from jax.experimental.pallas import tpu as pltpu
from jax.experimental.pallas import tpu_sc as plsc

N_NODES = 10000
N_IN = 128
HEADS = 8
CH = 16
HC = HEADS * CH
E_EDGES = 320000
E_TOT = E_EDGES + N_NODES

LANES = 16
NC = 2
NS = 16
NW = NC * NS

K_CHUNK = 64
EW = 10368
EP = EW * NW
N_CHUNKS = EW // K_CHUNK
NP = 10016
ROWS_PER_SUB = NP // NS

_SPLAT_DNUMS = lax.GatherDimensionNumbers(
    offset_dims=(), collapsed_slice_dims=(0,), start_index_map=(0,))


def _splat(vec, lane):
  idx = jnp.full((LANES, 1), lane, jnp.int32)
  return lax.gather(vec, idx, _SPLAT_DNUMS, (1,),
                    mode=lax.GatherScatterMode.PROMISE_IN_BOUNDS)


def _make_sc_edge_kernel(d_model, hs_w, hoff, heads):
  mesh = plsc.VectorSubcoreMesh(core_axis_name="c", subcore_axis_name="s")
  nch2 = N_CHUNKS // 2

  @functools.partial(
      pl.kernel,
      out_type=jax.ShapeDtypeStruct((NC, NP, hs_w), jnp.float32),
      mesh=mesh,
      compiler_params=pltpu.CompilerParams(use_tc_tiling_on_sc=False),
      scratch_types=[
          pltpu.VMEM((3, 2, K_CHUNK), jnp.int32),
          pltpu.VMEM((3, 2, K_CHUNK), jnp.int32),
          pltpu.VMEM((2, K_CHUNK, hs_w), jnp.float32),
          pltpu.VMEM((2, K_CHUNK, LANES), jnp.float32),
          pltpu.VMEM((2, K_CHUNK, hs_w), jnp.float32),
          pltpu.VMEM_SHARED((NP, hs_w), jnp.float32),
          pltpu.SemaphoreType.DMA,
          pltpu.SemaphoreType.DMA,
          pltpu.SemaphoreType.DMA,
          pltpu.SemaphoreType.DMA,
          pltpu.SemaphoreType.DMA,
      ],
  )
  def sc_kernel(hs_hbm, adst_hbm, src_hbm, dst_hbm, acc_out,
                sidx, didx, hsb, adb, mbw, acc_sh,
                gsem0, gsem1, ssem0, ssem1, isem):
    c = lax.axis_index("c")
    s = lax.axis_index("s")
    wid = s * NC + c
    wgroups = hs_w // LANES

    zero16 = jnp.zeros((LANES,), jnp.float32)

    def zero_body(i, _):
      for g in range(wgroups):
        mbw[0, i, g * LANES:(g + 1) * LANES] = zero16
      return 0

    lax.fori_loop(0, K_CHUNK, zero_body, 0)
    zbase = s * ROWS_PER_SUB
    for t in range(ROWS_PER_SUB // K_CHUNK):
      pltpu.sync_copy(mbw.at[0],
                      acc_sh.at[pl.ds(zbase + t * K_CHUNK, K_CHUNK)])
    rem = ROWS_PER_SUB % K_CHUNK
    if rem:
      rb = zbase + (ROWS_PER_SUB // K_CHUNK) * K_CHUNK
      pltpu.sync_copy(mbw.at[0, pl.ds(0, rem)], acc_sh.at[pl.ds(rb, rem)])
    plsc.subcore_barrier()

    def idx_descs(j, pp):
      row = wid * N_CHUNKS + 2 * j
      d1 = pltpu.make_async_copy(src_hbm.at[pl.ds(row, 2)], sidx.at[pp],
                                 isem)
      d2 = pltpu.make_async_copy(dst_hbm.at[pl.ds(row, 2)], didx.at[pp],
                                 isem)
      return d1, d2

    def gather_descs(pp, b):
      gs = gsem0 if b == 0 else gsem1
      g1 = pltpu.make_async_copy(hs_hbm.at[sidx.at[pp, b]], hsb.at[b], gs)
      g2 = pltpu.make_async_copy(adst_hbm.at[didx.at[pp, b]], adb.at[b], gs)
      return g1, g2

    def scatter_desc(pp, b):
      ss = ssem0 if b == 0 else ssem1
      return pltpu.make_async_copy(mbw.at[b], acc_sh.at[didx.at[pp, b]], ss)

    def compute(b):
      def edge_body(k4, _):
        for u in range(4):
          k = 4 * k4 + u
          a = hsb[b, k, hoff:hoff + LANES]
          bb = adb[b, k, :]
          e = a + bb
          e = jnp.maximum(e, 0.2 * e)
          w = jnp.exp(e)
          mbw[b, k, d_model:d_model + LANES] = w
          for h in range(heads):
            ws = _splat(w, h)
            mbw[b, k, h * LANES:(h + 1) * LANES] = (
                hsb[b, k, h * LANES:(h + 1) * LANES] * ws)
        return 0

      lax.fori_loop(0, K_CHUNK // 4, edge_body, 0)

    zero_j = jnp.int32(0)
    for d in idx_descs(zero_j, zero_j % 3):
      d.start()
      d.wait()
    for b in (0, 1):
      for g in gather_descs(zero_j % 3, b):
        g.start()

    def chunk_body(j, _):
      p0 = lax.rem(j, 3)
      p1 = lax.rem(j + 1, 3)
      pm1 = lax.rem(j + 2, 3)
      last = nch2 - 1

      @pl.when(j < last)
      def _():
        for d in idx_descs(j + 1, p1):
          d.start()

      for g in gather_descs(p0, 0):
        g.wait()

      @pl.when(j > 0)
      def _():
        scatter_desc(pm1, 0).wait()

      compute(0)
      pltpu.async_copy(mbw.at[0], acc_sh.at[didx.at[p0, 0]], ssem0,
                       add=True)

      @pl.when(j < last)
      def _():
        for d in idx_descs(j + 1, p1):
          d.wait()
        for g in gather_descs(p1, 0):
          g.start()

      for g in gather_descs(p0, 1):
        g.wait()

      @pl.when(j > 0)
      def _():
        scatter_desc(pm1, 1).wait()

      compute(1)
      pltpu.async_copy(mbw.at[1], acc_sh.at[didx.at[p0, 1]], ssem1,
                       add=True)

      @pl.when(j < last)
      def _():
        for g in gather_descs(p1, 1):
          g.start()

      return 0

    lax.fori_loop(0, nch2, chunk_body, 0)
    plast = jnp.int32((nch2 - 1) % 3)
    scatter_desc(plast, 0).wait()
    scatter_desc(plast, 1).wait()
    plsc.subcore_barrier()
    pltpu.sync_copy(acc_sh.at[pl.ds(zbase, ROWS_PER_SUB)],
                    acc_out.at[c, pl.ds(zbase, ROWS_PER_SUB)])

  return sc_kernel


_sc_edge_big = _make_sc_edge_kernel(HC, HC + LANES, HC, HEADS)
_sc_edge_small = _make_sc_edge_kernel(CH, 2 * CH, CH, 1)


def _group_mat(rows, cols):
  r = lax.broadcasted_iota(jnp.int32, (rows, cols), 0)
  c = lax.broadcasted_iota(jnp.int32, (rows, cols), 1)
  return (r // LANES == c).astype(jnp.float32)


def _group_mat_t(rows, cols):
  r = lax.broadcasted_iota(jnp.int32, (rows, cols), 0)
  c = lax.broadcasted_iota(jnp.int32, (rows, cols), 1)
  return (c // LANES == r).astype(jnp.float32)


def _dot(a, b):
  return jnp.dot(a, b, preferred_element_type=jnp.float32,
                 precision=lax.Precision.HIGHEST)


def _head_coeffs(h, af_src, af_dst):
  g = _group_mat(h.shape[1], LANES)
  return _dot(h * af_src, g), _dot(h * af_dst, g)


RB = 2000
N_RB = N_NODES // RB


def _tc1_body(x_ref, w1_ref, afs_ref, afd_ref, ws1_ref, bs1_ref,
              h1_ref, as16_ref, ad16_ref, xin_ref):
  x = x_ref[...]
  h1 = _dot(x, w1_ref[...])
  h1_ref[...] = h1
  as16, ad16 = _head_coeffs(h1, afs_ref[...], afd_ref[...])
  as16_ref[...] = as16
  ad16_ref[...] = ad16
  xin_ref[...] = _dot(x, ws1_ref[...]) + bs1_ref[...]


def _finalize_gat(acc_ref, b_ref):
  a = acc_ref[0] + acc_ref[1]
  d = a.shape[1] - LANES
  msg = a[:, :d]
  den16 = a[:, d:]
  dbn = _dot(den16, _group_mat_t(LANES, d))
  return msg / (dbn + 1e-16) + b_ref[...]


def _gat_stats_body(acc_ref, b_ref, gat_ref, sum_ref, sq_ref):
  gat = _finalize_gat(acc_ref, b_ref)
  gat_ref[...] = gat

  @pl.when(pl.program_id(0) == 0)
  def _init():
    sum_ref[...] = jnp.zeros_like(sum_ref)
    sq_ref[...] = jnp.zeros_like(sq_ref)

  sum_ref[0:1, :] += jnp.sum(gat, axis=0, keepdims=True)
  sq_ref[0:1, :] += jnp.sum(gat * gat, axis=0, keepdims=True)


def _bn_elu(gat_ref, sum_ref, sq_ref, g_ref, be_ref):
  mean = sum_ref[0:1, :] * (1.0 / N_NODES)
  var = sq_ref[0:1, :] * (1.0 / N_NODES) - mean * mean
  y = (gat_ref[...] - mean) * lax.rsqrt(var + 1e-5) * g_ref[...] + be_ref[...]
  return jnp.where(y > 0, y, jnp.exp(y) - 1.0)


def _tc2_body(gat_ref, sum_ref, sq_ref, g1_ref, be1_ref, xin_ref, w2_ref,
              afs_ref, afd_ref, ws2_ref, bs2_ref,
              h2_ref, as16_ref, ad16_ref, xskip_ref):
  t2 = _bn_elu(gat_ref, sum_ref, sq_ref, g1_ref, be1_ref) + xin_ref[...]
  h2 = _dot(t2, w2_ref[...])
  h2_ref[...] = h2
  as16, ad16 = _head_coeffs(h2, afs_ref[...], afd_ref[...])
  as16_ref[...] = as16
  ad16_ref[...] = ad16
  xskip_ref[...] = _dot(t2, ws2_ref[...]) + bs2_ref[...]


def _tc3_body(gat_ref, sum_ref, sq_ref, g2_ref, be2_ref, w3_ref,
              as3_ref, ad3_ref, h3_ref, as16_ref, ad16_ref):
  t3 = _bn_elu(gat_ref, sum_ref, sq_ref, g2_ref, be2_ref)
  h3 = _dot(t3, w3_ref[...])
  h3_ref[...] = h3
  lane0 = (lax.broadcasted_iota(jnp.int32, (CH, CH), 1) == 0
           ).astype(jnp.float32)
  as16_ref[...] = _dot(h3, jnp.reshape(as3_ref[...], (CH, 1)) * lane0)
  ad16_ref[...] = _dot(h3, jnp.reshape(ad3_ref[...], (CH, 1)) * lane0)


def _tc4_body(acc_ref, b3_ref, xskip_ref, out_ref):
  a = acc_ref[0] + acc_ref[1]
  den = a[:, CH:CH + 1]
  out_ref[...] = a[:, :CH] / (den + 1e-16) + b3_ref[...] + xskip_ref[...]


def _sds(shape):
  return jax.ShapeDtypeStruct(shape, jnp.float32)


def _pad_nodes(a):
  return jnp.pad(a, ((0, NP - N_NODES), (0, 0)))


def kernel(x, edge_index, W1, a_src1, a_dst1, b1, g1, be1,
           W2, a_src2, a_dst2, b2, g2, be2, W3, a_src3, a_dst3, b3,
           Ws1, bs1, Ws2, bs2):
  loop = jnp.arange(N_NODES, dtype=jnp.int32)
  pad = EP - E_TOT
  src = jnp.concatenate(
      [edge_index[0], loop, jnp.zeros((pad,), jnp.int32)]
  ).reshape(EP // K_CHUNK, K_CHUNK)
  dst = jnp.concatenate(
      [edge_index[1], loop, jnp.full((pad,), N_NODES, jnp.int32)]
  ).reshape(EP // K_CHUNK, K_CHUNK)

  r1 = lambda a: jnp.reshape(a, (1, -1))

  def rows(w):
    return pl.BlockSpec((RB, w), lambda i: (i, 0))

  def full(shape):
    nd = len(shape)
    return pl.BlockSpec(shape, lambda i: (0,) * nd)

  def parts(w):
    return pl.BlockSpec((NC, RB, w), lambda i: (0, i, 0))

  h1, as16, ad16, xin = pl.pallas_call(
      _tc1_body,
      grid=(N_RB,),
      in_specs=[rows(N_IN), full((N_IN, HC)), full((1, HC)), full((1, HC)),
                full((N_IN, HC)), full((1, HC))],
      out_specs=[rows(HC), rows(LANES), rows(LANES), rows(HC)],
      out_shape=[_sds((N_NODES, HC)), _sds((N_NODES, LANES)),
                 _sds((N_NODES, LANES)), _sds((N_NODES, HC))],
  )(x, W1, r1(a_src1), r1(a_dst1), Ws1, r1(bs1))

  hs1 = _pad_nodes(jnp.concatenate([h1, as16], axis=1))
  acc1 = _sc_edge_big(hs1, _pad_nodes(ad16), src, dst)

  def gat_stats(acc, b, w):
    return pl.pallas_call(
        _gat_stats_body,
        grid=(N_RB,),
        in_specs=[parts(w + LANES), full((1, w))],
        out_specs=[rows(w), full((8, w)), full((8, w))],
        out_shape=[_sds((N_NODES, w)), _sds((8, w)), _sds((8, w))],
    )(acc, b)

  gat1, sum1, sq1 = gat_stats(acc1, r1(b1), HC)

  h2, as16b, ad16b, xskip = pl.pallas_call(
      _tc2_body,
      grid=(N_RB,),
      in_specs=[rows(HC), full((8, HC)), full((8, HC)), full((1, HC)),
                full((1, HC)), rows(HC), full((HC, HC)), full((1, HC)),
                full((1, HC)), full((HC, CH)), full((1, CH))],
      out_specs=[rows(HC), rows(LANES), rows(LANES), rows(CH)],
      out_shape=[_sds((N_NODES, HC)), _sds((N_NODES, LANES)),
                 _sds((N_NODES, LANES)), _sds((N_NODES, CH))],
  )(gat1, sum1, sq1, r1(g1), r1(be1), xin, W2, r1(a_src2), r1(a_dst2),
    Ws2, r1(bs2))

  hs2 = _pad_nodes(jnp.concatenate([h2, as16b], axis=1))
  acc2 = _sc_edge_big(hs2, _pad_nodes(ad16b), src, dst)

  gat2, sum2, sq2 = gat_stats(acc2, r1(b2), HC)

  h3, as16c, ad16c = pl.pallas_call(
      _tc3_body,
      grid=(N_RB,),
      in_specs=[rows(HC), full((8, HC)), full((8, HC)), full((1, HC)),
                full((1, HC)), full((HC, CH)), full((1, CH)), full((1, CH))],
      out_specs=[rows(CH), rows(LANES), rows(LANES)],
      out_shape=[_sds((N_NODES, CH)), _sds((N_NODES, LANES)),
                 _sds((N_NODES, LANES))],
  )(gat2, sum2, sq2, r1(g2), r1(be2), W3, r1(a_src3), r1(a_dst3))

  hs3 = _pad_nodes(jnp.concatenate([h3, as16c], axis=1))
  acc3 = _sc_edge_small(hs3, _pad_nodes(ad16c), src, dst)

  out = pl.pallas_call(
      _tc4_body,
      grid=(N_RB,),
      in_specs=[parts(CH + LANES), full((1, CH)), rows(CH)],
      out_specs=rows(CH),
      out_shape=_sds((N_NODES, CH)),
  )(acc3, r1(b3), xskip)
  return out
